# baseline (speedup 1.0000x reference)
import functools

import jax
import jax.numpy as jnp
from jax.experimental import pallas as pl
from jax.experimental.pallas import tpu as pltpu

N = 50000
E = 800000
B = 512
MOL_IN = 128
DIM = 64
HEADS = 8
HD = HEADS * DIM

_BLK = 400


def _mm(a, b):
    return jax.lax.dot_general(a, b, (((1,), (0,)), ((), ())),
                               precision=jax.lax.Precision.HIGHEST)


def _front_body(x_ref, w0_ref, b0_ref, wg_ref, asrc_w_ref, adst_w_ref,
                h_ref, xg_ref, asrc_ref, adst_ref):
    x = x_ref[...]
    h = jnp.maximum(_mm(x, w0_ref[...]) + b0_ref[...][None, :], 0.0)
    h_ref[...] = h
    xg = _mm(h, wg_ref[...])
    xg_ref[...] = xg
    xg3 = xg.reshape(xg.shape[0], HEADS, DIM)
    asrc_ref[...] = (xg3 * asrc_w_ref[...][None]).sum(-1)
    adst_ref[...] = (xg3 * adst_w_ref[...][None]).sum(-1)


def _front(x, W0, b0, Wg, att_src, att_dst):
    grid = (N // _BLK,)
    out_shapes = (
        jax.ShapeDtypeStruct((N, DIM), jnp.float32),
        jax.ShapeDtypeStruct((N, HD), jnp.float32),
        jax.ShapeDtypeStruct((N, HEADS), jnp.float32),
        jax.ShapeDtypeStruct((N, HEADS), jnp.float32),
    )
    full = lambda s: pl.BlockSpec(s, lambda i: (0,) * len(s))
    return pl.pallas_call(
        _front_body,
        grid=grid,
        in_specs=[
            pl.BlockSpec((_BLK, MOL_IN), lambda i: (i, 0)),
            full((MOL_IN, DIM)),
            full((DIM,)),
            full((DIM, HD)),
            full((HEADS, DIM)),
            full((HEADS, DIM)),
        ],
        out_specs=(
            pl.BlockSpec((_BLK, DIM), lambda i: (i, 0)),
            pl.BlockSpec((_BLK, HD), lambda i: (i, 0)),
            pl.BlockSpec((_BLK, HEADS), lambda i: (i, 0)),
            pl.BlockSpec((_BLK, HEADS), lambda i: (i, 0)),
        ),
        out_shape=out_shapes,
    )(x, W0, b0, Wg, att_src, att_dst)


def _segment_softmax(e, seg, num_segments):
    m = jax.ops.segment_max(e, seg, num_segments=num_segments)
    m = jnp.where(jnp.isfinite(m), m, 0.0)
    ex = jnp.exp(e - m[seg])
    denom = jax.ops.segment_sum(ex, seg, num_segments=num_segments)
    return ex / (denom[seg] + 1e-16)


def kernel(x, edge_index, batch, W0, b0, Wg, att_src, att_dst, bg, Wh, bh,
           Wih, Whh, bih, bhh, W1, b1, W2, b2):
    n = x.shape[0]
    h, xg_flat, a_src, a_dst = _front(x, W0, b0, Wg, att_src, att_dst)
    xg = xg_flat.reshape(n, HEADS, DIM)

    loop = jnp.arange(n)
    src = jnp.concatenate([edge_index[0], loop])
    dst = jnp.concatenate([edge_index[1], loop])
    e = jax.nn.leaky_relu(a_src[src] + a_dst[dst], negative_slope=0.2)
    alpha = _segment_softmax(e, dst, n)
    msg = xg[src] * alpha[:, :, None]
    out = jax.ops.segment_sum(msg, dst, num_segments=n).reshape(n, HD) + bg
    out = jax.nn.relu(out)

    h2 = jax.nn.relu(out @ Wh + bh)

    q_star = jnp.zeros((B, 2 * DIM), h2.dtype)
    hs = jnp.zeros((B, DIM), h2.dtype)
    cs = jnp.zeros((B, DIM), h2.dtype)
    for _ in range(3):
        gates = q_star @ Wih.T + hs @ Whh.T + bih + bhh
        ig, fg, gg, og = jnp.split(gates, 4, axis=-1)
        cs = jax.nn.sigmoid(fg) * cs + jax.nn.sigmoid(ig) * jnp.tanh(gg)
        hs = jax.nn.sigmoid(og) * jnp.tanh(cs)
        q = hs
        eatt = (h2 * q[batch]).sum(-1)
        aatt = _segment_softmax(eatt, batch, B)
        r = jax.ops.segment_sum(aatt[:, None] * h2, batch, num_segments=B)
        q_star = jnp.concatenate([q, r], axis=-1)
    o = jax.nn.relu(q_star @ W1 + b1)
    o = o @ W2 + b2
    return o.reshape(-1)


# speedup vs baseline: 6.8014x; 6.8014x over previous
import functools

import jax
import jax.numpy as jnp
from jax import lax
from jax.experimental import pallas as pl
from jax.experimental.pallas import tpu as pltpu
from jax.experimental.pallas import tpu_sc as plsc

N = 50000
E = 800000
B = 512
MOL_IN = 128
DIM = 64
HEADS = 8
HD = HEADS * DIM

NC = 2
NS = 16
L = 16

NP = 50048
RPT = NP // NS
E2 = E + N
KB = 512
PTW = 26624
E2P = PTW * NC * NS
NB_W = PTW // KB
PTS = E2P // NS
NB_S = PTS // KB
EROWS = E2P // 128

_BLK = 400


def _mm(a, b):
    return lax.dot_general(a, b, (((1,), (0,)), ((), ())),
                           precision=lax.Precision.HIGHEST)


def _front_body(x_ref, w0_ref, b0_ref, wg_ref, asrc_w_ref, adst_w_ref,
                h_ref, xg_ref, asrc_ref, adst_ref, amax_ref):
    i = pl.program_id(0)
    x = x_ref[...]
    h = jnp.maximum(_mm(x, w0_ref[...]) + b0_ref[...][None, :], 0.0)
    h_ref[...] = h
    xg = _mm(h, wg_ref[...])
    xg_ref[...] = xg
    xg3 = xg.reshape(xg.shape[0], HEADS, DIM)
    a_s = (xg3 * asrc_w_ref[...][None]).sum(-1)
    a_d = (xg3 * adst_w_ref[...][None]).sum(-1)
    asrc_ref[...] = a_s
    adst_ref[...] = a_d
    bm = jnp.max(a_s, axis=0)

    @pl.when(i == 0)
    def _():
        amax_ref[...] = bm

    @pl.when(i > 0)
    def _():
        amax_ref[...] = jnp.maximum(amax_ref[...], bm)


def _front(x, W0, b0, Wg, att_src, att_dst):
    grid = (N // _BLK,)
    out_shapes = (
        jax.ShapeDtypeStruct((N, DIM), jnp.float32),
        jax.ShapeDtypeStruct((N, HD), jnp.float32),
        jax.ShapeDtypeStruct((N, HEADS), jnp.float32),
        jax.ShapeDtypeStruct((N, HEADS), jnp.float32),
        jax.ShapeDtypeStruct((HEADS,), jnp.float32),
    )
    full = lambda s: pl.BlockSpec(s, lambda i: (0,) * len(s))
    return pl.pallas_call(
        _front_body,
        grid=grid,
        in_specs=[
            pl.BlockSpec((_BLK, MOL_IN), lambda i: (i, 0)),
            full((MOL_IN, DIM)),
            full((DIM,)),
            full((DIM, HD)),
            full((HEADS, DIM)),
            full((HEADS, DIM)),
        ],
        out_specs=(
            pl.BlockSpec((_BLK, DIM), lambda i: (i, 0)),
            pl.BlockSpec((_BLK, HD), lambda i: (i, 0)),
            pl.BlockSpec((_BLK, HEADS), lambda i: (i, 0)),
            pl.BlockSpec((_BLK, HEADS), lambda i: (i, 0)),
            full((HEADS,)),
        ),
        out_shape=out_shapes,
    )(x, W0, b0, Wg, att_src, att_dst)


MB = 128


def _kw_body(as16, ad16, srcp, dstp, areph, z16,
             w8row, denomp,
             sidx, didx, g1, g2, wrow, arepv, dacc, sem):
    c = lax.axis_index("c")
    s = lax.axis_index("s")
    wid = c * NS + s
    pltpu.sync_copy(areph, arepv)
    pltpu.sync_copy(z16, dacc.at[pl.ds(pl.multiple_of(s * RPT, 8), RPT)])
    plsc.subcore_barrier()

    base = wid * PTW
    arep = arepv[...]

    def batch(b, carry):
        bb = pl.multiple_of(base + b * MB, MB)
        pltpu.sync_copy(srcp.at[pl.ds(bb, MB)], sidx)
        pltpu.sync_copy(dstp.at[pl.ds(bb, MB)], didx)
        pltpu.async_copy(as16.at[sidx], g1, sem).wait()
        pltpu.async_copy(ad16.at[didx], g2, sem).wait()

        def grp(k, carry2):
            v1 = g1[k, :]
            v2 = g2[k, :]
            z = v1 + v2
            zm = arep + v2
            w = jnp.exp(jnp.maximum(z, 0.2 * z) - jnp.maximum(zm, 0.2 * zm))
            wrow[k, :] = w
            return carry2

        lax.fori_loop(0, MB, grp, 0)
        pltpu.sync_copy(wrow, w8row.at[pl.ds(bb, MB)])
        pltpu.sync_copy(wrow, dacc.at[didx], add=True)
        return carry

    lax.fori_loop(0, PTW // MB, batch, 0)
    plsc.subcore_barrier()
    pltpu.sync_copy(dacc.at[pl.ds(pl.multiple_of(s * RPT, 8), RPT)],
                    denomp.at[pl.ds(pl.multiple_of(c * NP + s * RPT, 8), RPT)])


def _run_kw(as16, ad16, srcp, dstp, areph, z16):
    mesh = plsc.VectorSubcoreMesh(core_axis_name="c", subcore_axis_name="s",
                                  num_cores=NC, num_subcores=NS)
    f = pl.kernel(
        _kw_body,
        out_type=(
            jax.ShapeDtypeStruct((E2P, 16), jnp.float32),
            jax.ShapeDtypeStruct((NC * NP, 16), jnp.float32),
        ),
        mesh=mesh,
        compiler_params=pltpu.CompilerParams(use_tc_tiling_on_sc=False),
        scratch_types=[
            pltpu.VMEM((MB,), jnp.int32),
            pltpu.VMEM((MB,), jnp.int32),
            pltpu.VMEM((MB, 16), jnp.float32),
            pltpu.VMEM((MB, 16), jnp.float32),
            pltpu.VMEM((MB, 16), jnp.float32),
            pltpu.VMEM((L,), jnp.float32),
            pltpu.VMEM_SHARED((NP, 16), jnp.float32),
            pltpu.SemaphoreType.DMA,
        ],
    )
    return f(as16, ad16, srcp, dstp, areph, z16)


def _ks_body(srcp, dstp, w8row, xgt2, z32,
             sp2,
             sidx, didx, wb2, rows, acc, sem):
    c = lax.axis_index("c")
    s = lax.axis_index("s")
    tilebase = s * PTS

    for i in range(HEADS):
        xoff = (2 * i + c) * NP
        pltpu.sync_copy(z32, acc.at[pl.ds(pl.multiple_of(s * RPT, 8), RPT)])
        plsc.subcore_barrier()

        def batch(b, carry2, i=i, xoff=xoff):
            bb = pl.multiple_of(tilebase + b * MB, MB)
            pltpu.sync_copy(srcp.at[pl.ds(bb, MB)], sidx)
            pltpu.sync_copy(dstp.at[pl.ds(bb, MB)], didx)
            pltpu.sync_copy(w8row.at[pl.ds(bb, MB)], wb2)
            pltpu.async_copy(
                xgt2.at[pl.ds(pl.multiple_of(xoff, 8), NP)].at[sidx],
                rows, sem).wait()

            def scale(k, carry3, i=i):
                w = wb2[k, :][i]
                rows[k, pl.ds(0, 16)] = rows[k, pl.ds(0, 16)] * w
                rows[k, pl.ds(16, 16)] = rows[k, pl.ds(16, 16)] * w
                return carry3

            lax.fori_loop(0, MB, scale, 0)
            pltpu.sync_copy(rows, acc.at[didx], add=True)
            return carry2

        lax.fori_loop(0, PTS // MB, batch, 0)
        plsc.subcore_barrier()
        pltpu.sync_copy(acc.at[pl.ds(pl.multiple_of(s * RPT, 8), RPT)],
                        sp2.at[pl.ds(pl.multiple_of(xoff + s * RPT, 8), RPT)])
        plsc.subcore_barrier()


def _run_ks(srcp, dstp, w8row, xgt2, z32):
    mesh = plsc.VectorSubcoreMesh(core_axis_name="c", subcore_axis_name="s",
                                  num_cores=NC, num_subcores=NS)
    f = pl.kernel(
        _ks_body,
        out_type=jax.ShapeDtypeStruct((2 * HEADS * NP, 32), jnp.float32),
        mesh=mesh,
        compiler_params=pltpu.CompilerParams(use_tc_tiling_on_sc=False),
        scratch_types=[
            pltpu.VMEM((MB,), jnp.int32),
            pltpu.VMEM((MB,), jnp.int32),
            pltpu.VMEM((MB, 16), jnp.float32),
            pltpu.VMEM((MB, 32), jnp.float32),
            pltpu.VMEM_SHARED((NP, 32), jnp.float32),
            pltpu.SemaphoreType.DMA,
        ],
    )
    return f(srcp, dstp, w8row, xgt2, z32)


def _epi_body(s_ref, dn_ref, bg_ref, wh_ref, bh_ref, h2_ref):
    sblk = s_ref[...]
    inv = 1.0 / (dn_ref[...] + 1e-16)
    o = sblk.reshape(sblk.shape[0], HEADS, DIM) * inv[:, :, None]
    o = jnp.maximum(o.reshape(sblk.shape[0], HD) + bg_ref[...][None, :], 0.0)
    h2_ref[...] = jnp.maximum(_mm(o, wh_ref[...]) + bh_ref[...][None, :], 0.0)


def _epilogue(S, denom, bg, Wh, bh):
    grid = (N // _BLK,)
    full = lambda s: pl.BlockSpec(s, lambda i: (0,) * len(s))
    return pl.pallas_call(
        _epi_body,
        grid=grid,
        in_specs=[
            pl.BlockSpec((_BLK, HD), lambda i: (i, 0)),
            pl.BlockSpec((_BLK, HEADS), lambda i: (i, 0)),
            full((HD,)),
            full((HD, DIM)),
            full((DIM,)),
        ],
        out_specs=pl.BlockSpec((_BLK, DIM), lambda i: (i, 0)),
        out_shape=jax.ShapeDtypeStruct((N, DIM), jnp.float32),
    )(S, denom, bg, Wh, bh)


def _segment_softmax(e, seg, num_segments):
    m = jax.ops.segment_max(e, seg, num_segments=num_segments)
    m = jnp.where(jnp.isfinite(m), m, 0.0)
    ex = jnp.exp(e - m[seg])
    denom = jax.ops.segment_sum(ex, seg, num_segments=num_segments)
    return ex / (denom[seg] + 1e-16)


def kernel(x, edge_index, batch, W0, b0, Wg, att_src, att_dst, bg, Wh, bh,
           Wih, Whh, bih, bhh, W1, b1, W2, b2):
    h, xg_flat, a_src, a_dst, amax = _front(x, W0, b0, Wg, att_src, att_dst)

    loop = jnp.arange(N, dtype=jnp.int32)
    pad = E2P - E2
    srcp = jnp.concatenate([edge_index[0].astype(jnp.int32), loop,
                            jnp.zeros((pad,), jnp.int32)])
    dstp = jnp.concatenate([edge_index[1].astype(jnp.int32), loop,
                            jnp.full((pad,), NP - 1, jnp.int32)])
    as_p = jnp.pad(a_src, ((0, NP - N), (0, 0)))
    ad_p = jnp.pad(a_dst, ((0, NP - N), (0, 0)))
    as16 = jnp.concatenate([as_p, as_p], axis=1)
    ad16 = jnp.concatenate([ad_p, ad_p], axis=1)
    areph = jnp.tile(amax, 2)

    z16 = jnp.zeros((RPT, 16), jnp.float32)
    w8row, denomp = _run_kw(as16, ad16, srcp, dstp, areph, z16)

    z32 = jnp.zeros((RPT, 32), jnp.float32)
    xgt2 = (jnp.pad(xg_flat, ((0, NP - N), (0, 0)))
            .reshape(NP, 16, 32).transpose(1, 0, 2).reshape(16 * NP, 32))
    sp2 = _run_ks(srcp, dstp, w8row, xgt2, z32)
    S = (sp2.reshape(16, NP, 32).transpose(1, 0, 2).reshape(NP, HD))[:N]
    denom = denomp.reshape(NC, NP, 16)[:, :N, :HEADS].sum(0)

    h2 = _epilogue(S, denom, bg, Wh, bh)

    q_star = jnp.zeros((B, 2 * DIM), h2.dtype)
    hs = jnp.zeros((B, DIM), h2.dtype)
    cs = jnp.zeros((B, DIM), h2.dtype)
    for _ in range(3):
        gates = q_star @ Wih.T + hs @ Whh.T + bih + bhh
        ig, fg, gg, og = jnp.split(gates, 4, axis=-1)
        cs = jax.nn.sigmoid(fg) * cs + jax.nn.sigmoid(ig) * jnp.tanh(gg)
        hs = jax.nn.sigmoid(og) * jnp.tanh(cs)
        q = hs
        eatt = (h2 * q[batch]).sum(-1)
        aatt = _segment_softmax(eatt, batch, B)
        r = jax.ops.segment_sum(aatt[:, None] * h2, batch, num_segments=B)
        q_star = jnp.concatenate([q, r], axis=-1)
    o = jax.nn.relu(q_star @ W1 + b1)
    o = o @ W2 + b2
    return o.reshape(-1)


# speedup vs baseline: 8.6930x; 1.2781x over previous
import functools

import jax
import jax.numpy as jnp
from jax import lax
from jax.experimental import pallas as pl
from jax.experimental.pallas import tpu as pltpu
from jax.experimental.pallas import tpu_sc as plsc

N = 50000
E = 800000
B = 512
MOL_IN = 128
DIM = 64
HEADS = 8
HD = HEADS * DIM

NC = 2
NS = 16
L = 16

NP = 50048
RPT = NP // NS
E2 = E + N
KB = 512
PTW = 26624
E2P = PTW * NC * NS
NB_W = PTW // KB
PTS = E2P // NS
NB_S = PTS // KB
EROWS = E2P // 128

_BLK = 400


def _mm(a, b):
    return lax.dot_general(a, b, (((1,), (0,)), ((), ())),
                           precision=lax.Precision.HIGHEST)


def _front_body(x_ref, w0_ref, b0_ref, wg_ref, asrc_w_ref, adst_w_ref,
                h_ref, xg_ref, asrc_ref, adst_ref, amax_ref):
    i = pl.program_id(0)
    x = x_ref[...]
    h = jnp.maximum(_mm(x, w0_ref[...]) + b0_ref[...][None, :], 0.0)
    h_ref[...] = h
    xg = _mm(h, wg_ref[...])
    xg_ref[...] = xg
    xg3 = xg.reshape(xg.shape[0], HEADS, DIM)
    a_s = (xg3 * asrc_w_ref[...][None]).sum(-1)
    a_d = (xg3 * adst_w_ref[...][None]).sum(-1)
    asrc_ref[...] = a_s
    adst_ref[...] = a_d
    bm = jnp.max(a_s, axis=0)

    @pl.when(i == 0)
    def _():
        amax_ref[...] = bm

    @pl.when(i > 0)
    def _():
        amax_ref[...] = jnp.maximum(amax_ref[...], bm)


def _front(x, W0, b0, Wg, att_src, att_dst):
    grid = (N // _BLK,)
    out_shapes = (
        jax.ShapeDtypeStruct((N, DIM), jnp.float32),
        jax.ShapeDtypeStruct((N, HD), jnp.float32),
        jax.ShapeDtypeStruct((N, HEADS), jnp.float32),
        jax.ShapeDtypeStruct((N, HEADS), jnp.float32),
        jax.ShapeDtypeStruct((HEADS,), jnp.float32),
    )
    full = lambda s: pl.BlockSpec(s, lambda i: (0,) * len(s))
    return pl.pallas_call(
        _front_body,
        grid=grid,
        in_specs=[
            pl.BlockSpec((_BLK, MOL_IN), lambda i: (i, 0)),
            full((MOL_IN, DIM)),
            full((DIM,)),
            full((DIM, HD)),
            full((HEADS, DIM)),
            full((HEADS, DIM)),
        ],
        out_specs=(
            pl.BlockSpec((_BLK, DIM), lambda i: (i, 0)),
            pl.BlockSpec((_BLK, HD), lambda i: (i, 0)),
            pl.BlockSpec((_BLK, HEADS), lambda i: (i, 0)),
            pl.BlockSpec((_BLK, HEADS), lambda i: (i, 0)),
            full((HEADS,)),
        ),
        out_shape=out_shapes,
    )(x, W0, b0, Wg, att_src, att_dst)


MB = 256


def _kw_body(as16, ad16, srcp, dstp, areph, z16,
             w8row, denomp,
             sidx, didx, g1, g2, wrow, arepv, dacc, sem):
    c = lax.axis_index("c")
    s = lax.axis_index("s")
    wid = c * NS + s
    pltpu.sync_copy(areph, arepv)
    pltpu.sync_copy(z16, dacc.at[pl.ds(pl.multiple_of(s * RPT, 8), RPT)])
    plsc.subcore_barrier()

    base = wid * PTW
    arep = arepv[...]

    def batch(b, carry):
        bb = pl.multiple_of(base + b * MB, MB)
        pltpu.sync_copy(srcp.at[pl.ds(bb, MB)], sidx)
        pltpu.sync_copy(dstp.at[pl.ds(bb, MB)], didx)
        pltpu.async_copy(as16.at[sidx], g1, sem).wait()
        pltpu.async_copy(ad16.at[didx], g2, sem).wait()

        def grp(k, carry2):
            v1 = g1[k, :]
            v2 = g2[k, :]
            z = v1 + v2
            zm = arep + v2
            w = jnp.exp(jnp.maximum(z, 0.2 * z) - jnp.maximum(zm, 0.2 * zm))
            wrow[k, :] = w
            return carry2

        lax.fori_loop(0, MB, grp, 0)
        pltpu.sync_copy(wrow, w8row.at[pl.ds(bb, MB)])
        pltpu.sync_copy(wrow, dacc.at[didx], add=True)
        return carry

    lax.fori_loop(0, PTW // MB, batch, 0)
    plsc.subcore_barrier()
    pltpu.sync_copy(dacc.at[pl.ds(pl.multiple_of(s * RPT, 8), RPT)],
                    denomp.at[pl.ds(pl.multiple_of(c * NP + s * RPT, 8), RPT)])


def _run_kw(as16, ad16, srcp, dstp, areph, z16):
    mesh = plsc.VectorSubcoreMesh(core_axis_name="c", subcore_axis_name="s",
                                  num_cores=NC, num_subcores=NS)
    f = pl.kernel(
        _kw_body,
        out_type=(
            jax.ShapeDtypeStruct((E2P, 16), jnp.float32),
            jax.ShapeDtypeStruct((NC * NP, 16), jnp.float32),
        ),
        mesh=mesh,
        compiler_params=pltpu.CompilerParams(use_tc_tiling_on_sc=False),
        scratch_types=[
            pltpu.VMEM((MB,), jnp.int32),
            pltpu.VMEM((MB,), jnp.int32),
            pltpu.VMEM((MB, 16), jnp.float32),
            pltpu.VMEM((MB, 16), jnp.float32),
            pltpu.VMEM((MB, 16), jnp.float32),
            pltpu.VMEM((L,), jnp.float32),
            pltpu.VMEM_SHARED((NP, 16), jnp.float32),
            pltpu.SemaphoreType.DMA,
        ],
    )
    return f(as16, ad16, srcp, dstp, areph, z16)


def _ks_body(srcp, dstp, w8row, xgt2, z32,
             sp2,
             sidx, didx, wb2, rows, acc, sem):
    c = lax.axis_index("c")
    s = lax.axis_index("s")
    tilebase = s * PTS

    for i in range(HEADS):
        xoff = (2 * i + c) * NP
        pltpu.sync_copy(z32, acc.at[pl.ds(pl.multiple_of(s * RPT, 8), RPT)])
        plsc.subcore_barrier()

        def batch(b, carry2, i=i, xoff=xoff):
            bb = pl.multiple_of(tilebase + b * MB, MB)
            pltpu.sync_copy(srcp.at[pl.ds(bb, MB)], sidx)
            pltpu.sync_copy(dstp.at[pl.ds(bb, MB)], didx)
            pltpu.sync_copy(w8row.at[pl.ds(bb, MB)], wb2)
            pltpu.async_copy(
                xgt2.at[pl.ds(pl.multiple_of(xoff, 8), NP)].at[sidx],
                rows, sem).wait()

            def scale(k, carry3, i=i):
                w = wb2[k, :][i]
                rows[k, pl.ds(0, 16)] = rows[k, pl.ds(0, 16)] * w
                rows[k, pl.ds(16, 16)] = rows[k, pl.ds(16, 16)] * w
                return carry3

            lax.fori_loop(0, MB, scale, 0)
            pltpu.sync_copy(rows, acc.at[didx], add=True)
            return carry2

        lax.fori_loop(0, PTS // MB, batch, 0)
        plsc.subcore_barrier()
        pltpu.sync_copy(acc.at[pl.ds(pl.multiple_of(s * RPT, 8), RPT)],
                        sp2.at[pl.ds(pl.multiple_of(xoff + s * RPT, 8), RPT)])
        plsc.subcore_barrier()


def _run_ks(srcp, dstp, w8row, xgt2, z32):
    mesh = plsc.VectorSubcoreMesh(core_axis_name="c", subcore_axis_name="s",
                                  num_cores=NC, num_subcores=NS)
    f = pl.kernel(
        _ks_body,
        out_type=jax.ShapeDtypeStruct((2 * HEADS * NP, 32), jnp.float32),
        mesh=mesh,
        compiler_params=pltpu.CompilerParams(use_tc_tiling_on_sc=False),
        scratch_types=[
            pltpu.VMEM((MB,), jnp.int32),
            pltpu.VMEM((MB,), jnp.int32),
            pltpu.VMEM((MB, 16), jnp.float32),
            pltpu.VMEM((MB, 32), jnp.float32),
            pltpu.VMEM_SHARED((NP, 32), jnp.float32),
            pltpu.SemaphoreType.DMA,
        ],
    )
    return f(srcp, dstp, w8row, xgt2, z32)


def _epi_body(s_ref, dn_ref, bg_ref, wh_ref, bh_ref, h2_ref):
    sblk = s_ref[...]
    inv = 1.0 / (dn_ref[...] + 1e-16)
    o = sblk.reshape(sblk.shape[0], HEADS, DIM) * inv[:, :, None]
    o = jnp.maximum(o.reshape(sblk.shape[0], HD) + bg_ref[...][None, :], 0.0)
    h2_ref[...] = jnp.maximum(_mm(o, wh_ref[...]) + bh_ref[...][None, :], 0.0)


def _epilogue(S, denom, bg, Wh, bh):
    grid = (N // _BLK,)
    full = lambda s: pl.BlockSpec(s, lambda i: (0,) * len(s))
    return pl.pallas_call(
        _epi_body,
        grid=grid,
        in_specs=[
            pl.BlockSpec((_BLK, HD), lambda i: (i, 0)),
            pl.BlockSpec((_BLK, HEADS), lambda i: (i, 0)),
            full((HD,)),
            full((HD, DIM)),
            full((DIM,)),
        ],
        out_specs=pl.BlockSpec((_BLK, DIM), lambda i: (i, 0)),
        out_shape=jax.ShapeDtypeStruct((N, DIM), jnp.float32),
    )(S, denom, bg, Wh, bh)


def _segment_softmax(e, seg, num_segments):
    m = jax.ops.segment_max(e, seg, num_segments=num_segments)
    m = jnp.where(jnp.isfinite(m), m, 0.0)
    ex = jnp.exp(e - m[seg])
    denom = jax.ops.segment_sum(ex, seg, num_segments=num_segments)
    return ex / (denom[seg] + 1e-16)


def kernel(x, edge_index, batch, W0, b0, Wg, att_src, att_dst, bg, Wh, bh,
           Wih, Whh, bih, bhh, W1, b1, W2, b2):
    h, xg_flat, a_src, a_dst, amax = _front(x, W0, b0, Wg, att_src, att_dst)

    loop = jnp.arange(N, dtype=jnp.int32)
    pad = E2P - E2
    srcp = jnp.concatenate([edge_index[0].astype(jnp.int32), loop,
                            jnp.zeros((pad,), jnp.int32)])
    dstp = jnp.concatenate([edge_index[1].astype(jnp.int32), loop,
                            jnp.full((pad,), NP - 1, jnp.int32)])
    as_p = jnp.pad(a_src, ((0, NP - N), (0, 0)))
    ad_p = jnp.pad(a_dst, ((0, NP - N), (0, 0)))
    as16 = jnp.concatenate([as_p, as_p], axis=1)
    ad16 = jnp.concatenate([ad_p, ad_p], axis=1)
    areph = jnp.tile(amax, 2)

    z16 = jnp.zeros((RPT, 16), jnp.float32)
    w8row, denomp = _run_kw(as16, ad16, srcp, dstp, areph, z16)

    z32 = jnp.zeros((RPT, 32), jnp.float32)
    xgt2 = (jnp.pad(xg_flat, ((0, NP - N), (0, 0)))
            .reshape(NP, 16, 32).transpose(1, 0, 2).reshape(16 * NP, 32))
    sp2 = _run_ks(srcp, dstp, w8row, xgt2, z32)
    S = (sp2.reshape(16, NP, 32).transpose(1, 0, 2).reshape(NP, HD))[:N]
    denom = denomp.reshape(NC, NP, 16)[:, :N, :HEADS].sum(0)

    h2 = _epilogue(S, denom, bg, Wh, bh)

    q_star = jnp.zeros((B, 2 * DIM), h2.dtype)
    hs = jnp.zeros((B, DIM), h2.dtype)
    cs = jnp.zeros((B, DIM), h2.dtype)
    for _ in range(3):
        gates = q_star @ Wih.T + hs @ Whh.T + bih + bhh
        ig, fg, gg, og = jnp.split(gates, 4, axis=-1)
        cs = jax.nn.sigmoid(fg) * cs + jax.nn.sigmoid(ig) * jnp.tanh(gg)
        hs = jax.nn.sigmoid(og) * jnp.tanh(cs)
        q = hs
        eatt = (h2 * q[batch]).sum(-1)
        aatt = _segment_softmax(eatt, batch, B)
        r = jax.ops.segment_sum(aatt[:, None] * h2, batch, num_segments=B)
        q_star = jnp.concatenate([q, r], axis=-1)
    o = jax.nn.relu(q_star @ W1 + b1)
    o = o @ W2 + b2
    return o.reshape(-1)
